# baseline (speedup 1.0000x reference)
import jax
import jax.numpy as jnp
from jax.experimental import pallas as pl

_K = 64
_N = 32768
_ROWS = 128
_RPB = 8


def _topk_body(x_ref, out_ref):
    x = x_ref[...]
    col = jax.lax.broadcasted_iota(jnp.int32, x.shape, 1)
    acc0 = jnp.zeros((_RPB, _K), jnp.int32)
    acc_col = jax.lax.broadcasted_iota(jnp.int32, acc0.shape, 1)

    def step(j, carry):
        xc, acc = carry
        m = jnp.max(xc, axis=1, keepdims=True)
        cand = jnp.where(xc >= m, col, _N)
        fi = jnp.min(cand, axis=1, keepdims=True)
        acc = jnp.where(acc_col == j, fi, acc)
        xc = jnp.where(col == fi, -jnp.inf, xc)
        return xc, acc

    _, acc = jax.lax.fori_loop(0, _K, step, (x, acc0))
    out_ref[...] = acc


def kernel(x):
    return pl.pallas_call(
        _topk_body,
        grid=(_ROWS // _RPB,),
        in_specs=[pl.BlockSpec((_RPB, _N), lambda i: (i, 0))],
        out_specs=pl.BlockSpec((_RPB, _K), lambda i: (i, 0)),
        out_shape=jax.ShapeDtypeStruct((_ROWS, _K), jnp.int32),
    )(x)


# speedup vs baseline: 3.9219x; 3.9219x over previous
import functools

import jax
import jax.numpy as jnp
from jax import lax
from jax.experimental import pallas as pl
from jax.experimental.pallas import tpu as pltpu
from jax.experimental.pallas import tpu_sc as plsc

_K = 64
_N = 32768
_L = 16
_NV = _N // _L
_ROWS = 128
_NC = 2
_NS = 16
_NW = _NC * _NS
_RPW = _ROWS // _NW
_NHIST = 256 * _L
_MINKEY = -(2**31)


def _body(x_hbm, out_hbm, row_v, canda, candb, hist, merged, fin_i, outrow):
    wid = lax.axis_index("s") * _NC + lax.axis_index("c")
    lane = lax.iota(jnp.int32, _L)
    ones = jnp.ones((_L,), jnp.int32)
    zeros16 = jnp.zeros((_L,), jnp.int32)
    lane_base = lane * 256

    def clear_hist():
        def clr(i, c):
            hist[pl.ds(i * _L, _L)] = zeros16
            return c
        lax.fori_loop(0, _NHIST // _L, clr, 0)

    def merge_hist():
        def mrg(i, c):
            acc = zeros16
            for l in range(_L):
                acc = acc + hist[pl.ds(l * 256 + i * _L, _L)]
            merged[pl.ds(i * _L, _L)] = acc
            return c
        lax.fori_loop(0, 256 // _L, mrg, 0)

    def find_digit(need):
        def fd(j, carry):
            cum, d = carry
            jv = 15 - j
            vec = merged[pl.ds(jv * _L, _L)]
            suf = lax.rev(plsc.cumsum(lax.rev(vec, (0,))), (0,)) + cum
            cnt = jnp.sum((suf >= need).astype(jnp.int32))
            d = jnp.where((d < 0) & (cnt > 0), jv * _L + cnt - 1, d)
            return cum + jnp.sum(vec), d
        _, d = lax.fori_loop(0, 16, fd, (jnp.int32(0), jnp.int32(-1)))
        return d

    def do_row(r, carry):
        row = wid * _RPW + r
        pltpu.sync_copy(x_hbm.at[row], row_v)

        clear_hist()

        def sw_a(i, c):
            v = row_v[pl.ds(i * _L, _L)]
            bi = plsc.bitcast(v, jnp.int32)
            skey = jnp.where(bi < 0, bi ^ jnp.int32(0x7FFFFFFF), bi)
            row_v[pl.ds(i * _L, _L)] = plsc.bitcast(skey, jnp.float32)
            d = (skey >> 24) + 128
            plsc.addupdate_scatter(hist, [lane_base + d], ones)
            return c
        lax.fori_loop(0, _NV, sw_a, 0)
        merge_hist()
        d0 = find_digit(jnp.int32(_K))

        clear_hist()

        def sw_b(i, carry):
            nfin, ncand = carry
            skey = plsc.bitcast(row_v[pl.ds(i * _L, _L)], jnp.int32)
            d = (skey >> 24) + 128
            m_hi = d > d0
            m_eq = d == d0
            idx = lane + i * _L
            plsc.store_compressed(fin_i.at[pl.ds(nfin, _L)], idx, mask=m_hi)
            plsc.store_compressed(canda.at[pl.ds(ncand, _L)], idx, mask=m_eq)
            b1 = (skey >> 16) & 0xFF
            plsc.addupdate_scatter(hist, [lane_base + b1], ones, mask=m_eq)
            nfin = nfin + jnp.sum(m_hi.astype(jnp.int32))
            ncand = ncand + jnp.sum(m_eq.astype(jnp.int32))
            return nfin, ncand
        nfin, ncand = lax.fori_loop(0, _NV, sw_b, (jnp.int32(0), jnp.int32(0)))
        merge_hist()
        d1 = find_digit(_K - nfin)

        def sweep_list(src, n, shift, dcur, dst, do_hist, nfin):
            def body(i, carry):
                nfin, ndst = carry
                valid = (lane + i * _L) < n
                idx = src[pl.ds(i * _L, _L)]
                g = plsc.load_gather(row_v, [idx], mask=valid)
                skey = plsc.bitcast(g, jnp.int32)
                b = (skey >> shift) & 0xFF
                m_hi = (b > dcur) & valid
                m_eq = (b == dcur) & valid
                plsc.store_compressed(fin_i.at[pl.ds(nfin, _L)], idx, mask=m_hi)
                plsc.store_compressed(dst.at[pl.ds(ndst, _L)], idx, mask=m_eq)
                if do_hist:
                    b2 = (skey >> (shift - 8)) & 0xFF
                    plsc.addupdate_scatter(hist, [lane_base + b2], ones,
                                           mask=m_eq)
                nfin = nfin + jnp.sum(m_hi.astype(jnp.int32))
                ndst = ndst + jnp.sum(m_eq.astype(jnp.int32))
                return nfin, ndst
            nv = (n + _L - 1) // _L
            return lax.fori_loop(0, nv, body, (nfin, jnp.int32(0)))

        clear_hist()
        nfin, n2 = sweep_list(canda, ncand, 16, d1, candb, True, nfin)
        merge_hist()
        d2 = find_digit(_K - nfin)

        clear_hist()
        nfin, n3 = sweep_list(candb, n2, 8, d2, canda, True, nfin)
        merge_hist()
        d3 = find_digit(_K - nfin)

        nfin, n_eq = sweep_list(canda, n3, 0, d3, candb, False, nfin)

        need_eq = _K - nfin

        def app(i, nf):
            valid = (lane + i * _L) < need_eq
            idxv = candb[pl.ds(i * _L, _L)]
            plsc.store_compressed(fin_i.at[pl.ds(nf, _L)], idxv, mask=valid)
            return nf + jnp.sum(valid.astype(jnp.int32))
        lax.fori_loop(0, (need_eq + _L - 1) // _L, app, nfin)

        ks = []
        for j in range(4):
            fi = fin_i[pl.ds(j * _L, _L)]
            ks.append(plsc.bitcast(plsc.load_gather(row_v, [fi]), jnp.int32))

        def sel(j, kvec):
            k0, k1, k2, k3 = kvec
            g = jnp.max(jnp.maximum(jnp.maximum(k0, k1),
                                    jnp.maximum(k2, k3)))
            posv = zeros16 + jnp.int32(9999)
            for jj, kj in enumerate((k0, k1, k2, k3)):
                f = plsc.all_reduce_ffs(kj == g)
                posv = jnp.minimum(posv,
                                   jnp.where(f < _L, f + jj * _L, 9999))
            iv = plsc.load_gather(fin_i, [posv])
            plsc.store_scatter(outrow, [zeros16 + j], iv, mask=lane == 0)
            out = []
            for jj, kj in enumerate((k0, k1, k2, k3)):
                out.append(jnp.where(posv - jj * _L == lane, _MINKEY, kj))
            return tuple(out)
        lax.fori_loop(0, _K, sel, tuple(ks))

        pltpu.sync_copy(outrow, out_hbm.at[row])
        return carry

    lax.fori_loop(0, _RPW, do_row, 0)


@jax.jit
def kernel(x):
    f = pl.kernel(
        _body,
        out_type=jax.ShapeDtypeStruct((_ROWS, _K), jnp.int32),
        mesh=plsc.VectorSubcoreMesh(core_axis_name="c", subcore_axis_name="s",
                                    num_cores=_NC, num_subcores=_NS),
        compiler_params=pltpu.CompilerParams(needs_layout_passes=False),
        scratch_types=[
            pltpu.VMEM((_N,), jnp.float32),
            pltpu.VMEM((_N,), jnp.int32),
            pltpu.VMEM((_N,), jnp.int32),
            pltpu.VMEM((_NHIST,), jnp.int32),
            pltpu.VMEM((256,), jnp.int32),
            pltpu.VMEM((_K + _L,), jnp.int32),
            pltpu.VMEM((_K,), jnp.int32),
        ],
    )
    return f(x)
